# baseline (speedup 1.0000x reference)
import jax
import jax.numpy as jnp
from jax import lax
from jax.experimental import pallas as pl
from jax.experimental.pallas import tpu as pltpu
from jax.experimental.pallas import tpu_sc as plsc

B = 4096 * 200
D = 64
NW = 32
BPW = B // NW
C = 128
NCHUNK = BPW // C
NB = 4
NS = 2
LA = 2
SCALE = 8.0


def _body(idx_hbm, table_hbm, out_hbm, idx_v, wide_v, stg_v, *sems):
    isems = sems[0:NB]
    gsems = sems[NB:2 * NB]
    ssems = sems[2 * NB:2 * NB + NS]
    wid = lax.axis_index("s") * 2 + lax.axis_index("c")
    base = wid * BPW

    def issue_idx(g, b):
        off = pl.multiple_of(base + g * C, C)
        pltpu.async_copy(idx_hbm.at[pl.ds(off, C)], idx_v.at[b], isems[b])

    def wait_idx(b):
        off = pl.multiple_of(base, C)
        pltpu.make_async_copy(idx_hbm.at[pl.ds(off, C)], idx_v.at[b],
                              isems[b]).wait()

    def issue_gather(g, b):
        pltpu.async_copy(table_hbm.at[idx_v.at[b]], wide_v.at[b], gsems[b])

    def wait_gather(b):
        pltpu.make_async_copy(table_hbm.at[idx_v.at[b]], wide_v.at[b],
                              gsems[b]).wait()

    def issue_scatter(g, b):
        off = pl.multiple_of((base + g * C) // 2, C // 2)
        pltpu.async_copy(stg_v.at[b], out_hbm.at[pl.ds(off, C // 2)],
                         ssems[b])

    def wait_scatter(b):
        off = pl.multiple_of(base // 2, C // 2)
        pltpu.make_async_copy(stg_v.at[b], out_hbm.at[pl.ds(off, C // 2)],
                              ssems[b]).wait()

    for g in range(LA + 1):
        issue_idx(g, g % NB)
    for g in range(LA):
        wait_idx(g % NB)
        issue_gather(g, g % NB)

    @pl.loop(0, NCHUNK, step=NB)
    def _(t):
        for b in range(NB):
            g = t + b
            wait_gather(b)

            bi = (b + LA + 1) % NB
            bg = (b + LA) % NB
            bs = b % NS

            @pl.when(g + LA + 1 < NCHUNK)
            def _():
                issue_idx(g + LA + 1, bi)

            @pl.when(g + LA < NCHUNK)
            def _():
                wait_idx(bg)
                issue_gather(g + LA, bg)

            @pl.when(g >= NS)
            def _():
                wait_scatter(bs)

            @plsc.parallel_loop(0, C // 2, step=1, unroll=2)
            def _(k):
                for j in range(D // 16):
                    sl = pl.ds(j * 16, 16)
                    stg_v[bs, k, sl] = wide_v[b, 2 * k, sl] * SCALE
                for j in range(D // 16):
                    stg_v[bs, k, pl.ds(D + j * 16, 16)] = (
                        wide_v[b, 2 * k + 1, pl.ds(j * 16, 16)] * SCALE)

            issue_scatter(g, bs)

    for b in range(NS):
        wait_scatter(b)


NBATCH = 4096
SEQ = 200
TBB = 128


def _tc_transpose_body(pairs_ref, out_ref):
    def q_step(q, carry):
        for h in range(2):
            v = pairs_ref[:, q, pl.ds(h * D, D)]
            out_ref[2 * q + h] = jnp.transpose(v, (1, 0))
        return carry

    lax.fori_loop(0, SEQ // 2, q_step, 0, unroll=2)


def kernel(x, table):
    xf = x.reshape(-1).astype(jnp.int32)
    table_p = jnp.pad(table, ((0, 0), (0, D)))
    pairs = pl.kernel(
        _body,
        mesh=plsc.VectorSubcoreMesh(core_axis_name="c", subcore_axis_name="s"),
        compiler_params=pltpu.CompilerParams(use_tc_tiling_on_sc=True),
        out_type=jax.ShapeDtypeStruct((B // 2, 2 * D), jnp.float32),
        scratch_types=[
            pltpu.VMEM((NB, C), jnp.int32),
            pltpu.VMEM((NB, C, 2 * D), jnp.float32),
            pltpu.VMEM((NS, C // 2, 2 * D), jnp.float32),
        ] + [pltpu.SemaphoreType.DMA] * (2 * NB + NS),
    )(xf, table_p)
    pairs3 = pairs.reshape(NBATCH, SEQ // 2, 2 * D)
    o3 = pl.pallas_call(
        _tc_transpose_body,
        grid=(NBATCH // TBB,),
        in_specs=[pl.BlockSpec((TBB, SEQ // 2, 2 * D), lambda i: (i, 0, 0))],
        out_specs=pl.BlockSpec((SEQ, D, TBB), lambda i: (0, 0, i)),
        out_shape=jax.ShapeDtypeStruct((SEQ, D, NBATCH), jnp.float32),
    )(pairs3)
    return o3.transpose(2, 0, 1)


# speedup vs baseline: 1.1927x; 1.1927x over previous
import jax
import jax.numpy as jnp
from jax import lax
from jax.experimental import pallas as pl
from jax.experimental.pallas import tpu as pltpu
from jax.experimental.pallas import tpu_sc as plsc

B = 4096 * 200
D = 64
NW = 32
BPW = B // NW
NBATCH = 4096
SEQ = 200
C = SEQ
NCHUNK = BPW // C
NB = 2
NS = 2
G0 = 96
G1 = C - G0
SCALE = 8.0
TBB = 128


def _sc_body(idx_hbm, table_hbm, out_hbm, idx_v, wide_v, stg_v, *sems):
    isems = sems[0:NB]
    gsems = sems[NB:2 * NB]
    ssems = sems[2 * NB:2 * NB + NS]
    wid = lax.axis_index("s") * 2 + lax.axis_index("c")
    base = wid * BPW
    brow = wid * NCHUNK

    def issue_idx(g, b):
        off = pl.multiple_of(base + g * C, C)
        pltpu.async_copy(idx_hbm.at[pl.ds(off, C)],
                         idx_v.at[pl.ds(b * C, C)], isems[b])

    def wait_idx(b):
        off = pl.multiple_of(base, C)
        pltpu.make_async_copy(idx_hbm.at[pl.ds(off, C)],
                              idx_v.at[pl.ds(b * C, C)], isems[b]).wait()

    def issue_gather(g, b):
        pltpu.async_copy(table_hbm.at[idx_v.at[pl.ds(b * C, G0)]],
                         wide_v.at[b, pl.ds(0, G0)], gsems[b])
        pltpu.async_copy(table_hbm.at[idx_v.at[pl.ds(b * C + G0, G1)]],
                         wide_v.at[b, pl.ds(G0, G1)], gsems[b])

    def wait_gather(b):
        pltpu.make_async_copy(table_hbm.at[idx_v.at[pl.ds(b * C, G0)]],
                              wide_v.at[b, pl.ds(0, G0)], gsems[b]).wait()
        pltpu.make_async_copy(table_hbm.at[idx_v.at[pl.ds(b * C + G0, G1)]],
                              wide_v.at[b, pl.ds(G0, G1)], gsems[b]).wait()

    def issue_scatter(g, b):
        pltpu.async_copy(stg_v.at[b], out_hbm.at[pl.ds(brow + g, 1)],
                         ssems[b])

    def wait_scatter(b):
        pltpu.make_async_copy(stg_v.at[b], out_hbm.at[pl.ds(brow, 1)],
                              ssems[b]).wait()

    issue_idx(0, 0)
    issue_idx(1, 1)
    wait_idx(0)
    issue_gather(0, 0)

    @pl.loop(0, NCHUNK, step=NB)
    def _(t):
        for b in range(NB):
            g = t + b
            wait_gather(b)

            @pl.when(g + NB < NCHUNK)
            def _():
                issue_idx(g + NB, b)

            @pl.when(g + 1 < NCHUNK)
            def _():
                wait_idx((b + 1) % NB)
                issue_gather(g + 1, (b + 1) % NB)

            bs = b % NS

            @pl.when(g >= NS)
            def _():
                wait_scatter(bs)

            @plsc.parallel_loop(0, C // 2, step=1, unroll=2)
            def _(k):
                for j in range(D // 16):
                    sl = pl.ds(j * 16, 16)
                    stg_v[bs, 0, k, sl] = wide_v[b, 2 * k, sl] * SCALE
                for j in range(D // 16):
                    stg_v[bs, 0, k, pl.ds(D + j * 16, 16)] = (
                        wide_v[b, 2 * k + 1, pl.ds(j * 16, 16)] * SCALE)

            issue_scatter(g, bs)

    for b in range(NS):
        wait_scatter(b)


def _tc_body(pairs_ref, out_ref):
    def q_step(q, carry):
        for h in range(2):
            v = pairs_ref[:, q, pl.ds(h * D, D)]
            out_ref[2 * q + h] = jnp.transpose(v, (1, 0))
        return carry

    lax.fori_loop(0, SEQ // 2, q_step, 0, unroll=4)


def kernel(x, table):
    xf = x.reshape(-1).astype(jnp.int32)
    table_p = jnp.pad(table, ((0, 0), (0, D)))
    pairs3 = pl.kernel(
        _sc_body,
        mesh=plsc.VectorSubcoreMesh(core_axis_name="c", subcore_axis_name="s"),
        compiler_params=pltpu.CompilerParams(use_tc_tiling_on_sc=True),
        out_type=jax.ShapeDtypeStruct((NBATCH, SEQ // 2, 2 * D),
                                      jnp.float32),
        scratch_types=[
            pltpu.VMEM((NB * C,), jnp.int32),
            pltpu.VMEM((NB, C, 2 * D), jnp.float32),
            pltpu.VMEM((NS, 1, C // 2, 2 * D), jnp.float32),
        ] + [pltpu.SemaphoreType.DMA] * (2 * NB + NS),
    )(xf, table_p)
    o3 = pl.pallas_call(
        _tc_body,
        grid=(NBATCH // TBB,),
        in_specs=[pl.BlockSpec((TBB, SEQ // 2, 2 * D), lambda i: (i, 0, 0))],
        out_specs=pl.BlockSpec((SEQ, D, TBB), lambda i: (0, 0, i)),
        out_shape=jax.ShapeDtypeStruct((SEQ, D, NBATCH), jnp.float32),
    )(pairs3)
    return o3.transpose(2, 0, 1)


# speedup vs baseline: 1.2129x; 1.0169x over previous
import jax
import jax.numpy as jnp
from jax import lax
from jax.experimental import pallas as pl
from jax.experimental.pallas import tpu as pltpu
from jax.experimental.pallas import tpu_sc as plsc

B = 4096 * 200
D = 64
NW = 32
BPW = B // NW
NBATCH = 4096
SEQ = 200
C = SEQ
NCHUNK = BPW // C
NB = 2
NS = 2
G0 = 96
G1 = C - G0
SCALE = 8.0
TBB = 256


def _sc_body(idx_hbm, table_hbm, out_hbm, idx_v, wide_v, stg_v, *sems):
    isems = sems[0:NB]
    gsems = sems[NB:2 * NB]
    ssems = sems[2 * NB:2 * NB + NS]
    wid = lax.axis_index("s") * 2 + lax.axis_index("c")
    base = wid * BPW
    brow = wid * NCHUNK

    def issue_idx(g, b):
        off = pl.multiple_of(base + g * C, C)
        pltpu.async_copy(idx_hbm.at[pl.ds(off, C)],
                         idx_v.at[pl.ds(b * C, C)], isems[b])

    def wait_idx(b):
        off = pl.multiple_of(base, C)
        pltpu.make_async_copy(idx_hbm.at[pl.ds(off, C)],
                              idx_v.at[pl.ds(b * C, C)], isems[b]).wait()

    def issue_gather(g, b):
        pltpu.async_copy(table_hbm.at[idx_v.at[pl.ds(b * C, G0)]],
                         wide_v.at[b, pl.ds(0, G0)], gsems[b])
        pltpu.async_copy(table_hbm.at[idx_v.at[pl.ds(b * C + G0, G1)]],
                         wide_v.at[b, pl.ds(G0, G1)], gsems[b])

    def wait_gather(b):
        pltpu.make_async_copy(table_hbm.at[idx_v.at[pl.ds(b * C, G0)]],
                              wide_v.at[b, pl.ds(0, G0)], gsems[b]).wait()
        pltpu.make_async_copy(table_hbm.at[idx_v.at[pl.ds(b * C + G0, G1)]],
                              wide_v.at[b, pl.ds(G0, G1)], gsems[b]).wait()

    def issue_scatter(g, b):
        pltpu.async_copy(stg_v.at[b], out_hbm.at[pl.ds(brow + g, 1)],
                         ssems[b])

    def wait_scatter(b):
        pltpu.make_async_copy(stg_v.at[b], out_hbm.at[pl.ds(brow, 1)],
                              ssems[b]).wait()

    issue_idx(0, 0)
    issue_idx(1, 1)
    wait_idx(0)
    issue_gather(0, 0)

    @pl.loop(0, NCHUNK, step=NB)
    def _(t):
        for b in range(NB):
            g = t + b
            wait_gather(b)

            @pl.when(g + NB < NCHUNK)
            def _():
                issue_idx(g + NB, b)

            @pl.when(g + 1 < NCHUNK)
            def _():
                wait_idx((b + 1) % NB)
                issue_gather(g + 1, (b + 1) % NB)

            bs = b % NS

            @pl.when(g >= NS)
            def _():
                wait_scatter(bs)

            @plsc.parallel_loop(0, C // 2, step=1, unroll=2)
            def _(k):
                for j in range(D // 16):
                    sl = pl.ds(j * 16, 16)
                    stg_v[bs, 0, k, sl] = wide_v[b, 2 * k, sl] * SCALE
                for j in range(D // 16):
                    stg_v[bs, 0, k, pl.ds(D + j * 16, 16)] = (
                        wide_v[b, 2 * k + 1, pl.ds(j * 16, 16)] * SCALE)

            issue_scatter(g, bs)

    for b in range(NS):
        wait_scatter(b)


def _tc_body(pairs_ref, out_ref):
    def q_step(q, carry):
        for h in range(2):
            v = pairs_ref[:, q, pl.ds(h * D, D)]
            out_ref[2 * q + h] = jnp.transpose(v, (1, 0))
        return carry

    lax.fori_loop(0, SEQ // 2, q_step, 0, unroll=8)


def kernel(x, table):
    xf = x.reshape(-1).astype(jnp.int32)
    table_p = jnp.pad(table, ((0, 0), (0, D)))
    pairs3 = pl.kernel(
        _sc_body,
        mesh=plsc.VectorSubcoreMesh(core_axis_name="c", subcore_axis_name="s"),
        compiler_params=pltpu.CompilerParams(use_tc_tiling_on_sc=True),
        out_type=jax.ShapeDtypeStruct((NBATCH, SEQ // 2, 2 * D),
                                      jnp.float32),
        scratch_types=[
            pltpu.VMEM((NB * C,), jnp.int32),
            pltpu.VMEM((NB, C, 2 * D), jnp.float32),
            pltpu.VMEM((NS, 1, C // 2, 2 * D), jnp.float32),
        ] + [pltpu.SemaphoreType.DMA] * (2 * NB + NS),
    )(xf, table_p)
    o3 = pl.pallas_call(
        _tc_body,
        grid=(NBATCH // TBB,),
        in_specs=[pl.BlockSpec((TBB, SEQ // 2, 2 * D), lambda i: (i, 0, 0))],
        out_specs=pl.BlockSpec((SEQ, D, TBB), lambda i: (0, 0, i)),
        out_shape=jax.ShapeDtypeStruct((SEQ, D, NBATCH), jnp.float32),
    )(pairs3)
    return o3.transpose(2, 0, 1)


# speedup vs baseline: 1.2596x; 1.0386x over previous
import jax
import jax.numpy as jnp
from jax import lax
from jax.experimental import pallas as pl
from jax.experimental.pallas import tpu as pltpu
from jax.experimental.pallas import tpu_sc as plsc

B = 4096 * 200
D = 64
NW = 32
BPW = B // NW
NBATCH = 4096
SEQ = 200
C = SEQ
NCHUNK = BPW // C
NB = 2
NS = 2
G0 = 96
G1 = C - G0
SCALE = 8.0
TBB = 256


def _sc_body(idx_hbm, table_hbm, out_hbm, idx_v, wide_v, stg_v, *sems):
    isems = sems[0:NB]
    gsems = sems[NB:2 * NB]
    ssems = sems[2 * NB:2 * NB + NS]
    wid = lax.axis_index("s") * 2 + lax.axis_index("c")
    base = wid * BPW
    brow = wid * NCHUNK

    def issue_idx(g, b):
        off = pl.multiple_of(base + g * C, C)
        pltpu.async_copy(idx_hbm.at[pl.ds(off, C)],
                         idx_v.at[pl.ds(b * C, C)], isems[b])

    def wait_idx(b):
        off = pl.multiple_of(base, C)
        pltpu.make_async_copy(idx_hbm.at[pl.ds(off, C)],
                              idx_v.at[pl.ds(b * C, C)], isems[b]).wait()

    def issue_gather(g, b):
        pltpu.async_copy(table_hbm.at[idx_v.at[pl.ds(b * C, G0)]],
                         wide_v.at[b, pl.ds(0, G0)], gsems[b])
        pltpu.async_copy(table_hbm.at[idx_v.at[pl.ds(b * C + G0, G1)]],
                         wide_v.at[b, pl.ds(G0, G1)], gsems[b])

    def wait_gather(b):
        pltpu.make_async_copy(table_hbm.at[idx_v.at[pl.ds(b * C, G0)]],
                              wide_v.at[b, pl.ds(0, G0)], gsems[b]).wait()
        pltpu.make_async_copy(table_hbm.at[idx_v.at[pl.ds(b * C + G0, G1)]],
                              wide_v.at[b, pl.ds(G0, G1)], gsems[b]).wait()

    def issue_scatter(g, b):
        pltpu.async_copy(stg_v.at[b], out_hbm.at[pl.ds(brow + g, 1)],
                         ssems[b])

    def wait_scatter(b):
        pltpu.make_async_copy(stg_v.at[b], out_hbm.at[pl.ds(brow, 1)],
                              ssems[b]).wait()

    issue_idx(0, 0)
    issue_idx(1, 1)
    wait_idx(0)
    issue_gather(0, 0)

    @pl.loop(0, NCHUNK, step=NB)
    def _(t):
        for b in range(NB):
            g = t + b

            @pl.when(g + 1 < NCHUNK)
            def _():
                wait_idx((b + 1) % NB)
                issue_gather(g + 1, (b + 1) % NB)

            wait_gather(b)

            @pl.when(g + NB < NCHUNK)
            def _():
                issue_idx(g + NB, b)

            bs = b % NS

            @pl.when(g >= NS)
            def _():
                wait_scatter(bs)

            @plsc.parallel_loop(0, C // 2, step=1, unroll=2)
            def _(k):
                for j in range(D // 16):
                    sl = pl.ds(j * 16, 16)
                    stg_v[bs, 0, k, sl] = wide_v[b, 2 * k, sl] * SCALE
                for j in range(D // 16):
                    stg_v[bs, 0, k, pl.ds(D + j * 16, 16)] = (
                        wide_v[b, 2 * k + 1, pl.ds(j * 16, 16)] * SCALE)

            issue_scatter(g, bs)

    for b in range(NS):
        wait_scatter(b)


def _tc_body(pairs_ref, out_ref):
    def q_step(q, carry):
        for h in range(2):
            v = pairs_ref[:, q, pl.ds(h * D, D)]
            out_ref[2 * q + h] = jnp.transpose(v, (1, 0))
        return carry

    lax.fori_loop(0, SEQ // 2, q_step, 0, unroll=8)


def kernel(x, table):
    xf = x.reshape(-1).astype(jnp.int32)
    table_p = jnp.pad(table, ((0, 0), (0, D)))
    pairs3 = pl.kernel(
        _sc_body,
        mesh=plsc.VectorSubcoreMesh(core_axis_name="c", subcore_axis_name="s"),
        compiler_params=pltpu.CompilerParams(use_tc_tiling_on_sc=True),
        out_type=jax.ShapeDtypeStruct((NBATCH, SEQ // 2, 2 * D),
                                      jnp.float32),
        scratch_types=[
            pltpu.VMEM((NB * C,), jnp.int32),
            pltpu.VMEM((NB, C, 2 * D), jnp.float32),
            pltpu.VMEM((NS, 1, C // 2, 2 * D), jnp.float32),
        ] + [pltpu.SemaphoreType.DMA] * (2 * NB + NS),
    )(xf, table_p)
    o3 = pl.pallas_call(
        _tc_body,
        grid=(NBATCH // TBB,),
        in_specs=[pl.BlockSpec((TBB, SEQ // 2, 2 * D), lambda i: (i, 0, 0))],
        out_specs=pl.BlockSpec((SEQ, D, TBB), lambda i: (0, 0, i)),
        out_shape=jax.ShapeDtypeStruct((SEQ, D, NBATCH), jnp.float32),
    )(pairs3)
    return o3.transpose(2, 0, 1)
